# Initial kernel scaffold; baseline (speedup 1.0000x reference)
import jax
import jax.numpy as jnp
from jax.experimental import pallas as pl


def kernel(tokens, embedding):
    raise NotImplementedError("write your pallas kernel here")


# speedup vs baseline: 2.8224x; 2.8224x over previous
import functools
import math

import jax
import jax.numpy as jnp
from jax import lax
from jax.experimental import pallas as pl
from jax.experimental.pallas import tpu as pltpu
from jax.experimental.pallas import tpu_sc as plsc

EMB_D = 128
SCALE = math.sqrt(float(EMB_D))

NUM_CORES = 2
NUM_SUBCORES = 16
NUM_WORKERS = NUM_CORES * NUM_SUBCORES
TOKENS_TOTAL = 4096 * 50
PER_WORKER = TOKENS_TOTAL // NUM_WORKERS
CHUNK = 128
NUM_CHUNKS = PER_WORKER // CHUNK
LANES = 16
VECS_PER_ROW = EMB_D // LANES


def _emb_body(table_hbm, tok_hbm, out_hbm, idx_v, buf0, buf1,
              gs0, gs1, os0, os1):
    wid = lax.axis_index("s") * NUM_CORES + lax.axis_index("c")
    bufs = (buf0, buf1)
    gsems = (gs0, gs1)
    osems = (os0, os1)

    pltpu.sync_copy(tok_hbm.at[wid], idx_v)

    def start_gather(j, slot):
        return pltpu.async_copy(
            table_hbm.at[idx_v.at[j]], bufs[slot], gsems[slot])

    gh = [None, None]
    oh = [None, None]
    gh[0] = start_gather(0, 0)
    for j in range(NUM_CHUNKS):
        b = j & 1
        nb = b ^ 1
        if j + 1 < NUM_CHUNKS:
            if oh[nb] is not None:
                oh[nb].wait()
                oh[nb] = None
            gh[nb] = start_gather(j + 1, nb)
        gh[b].wait()
        buf = bufs[b]

        @plsc.parallel_loop(0, CHUNK, 1, unroll=4)
        def _scale_row(i):
            for k in range(VECS_PER_ROW):
                sl = pl.ds(k * LANES, LANES)
                buf[i, sl] = buf[i, sl] * SCALE

        oh[b] = pltpu.async_copy(buf, out_hbm.at[wid, j], osems[b])
    if oh[0] is not None:
        oh[0].wait()
    if oh[1] is not None:
        oh[1].wait()


@functools.partial(jax.jit, static_argnames=())
def _emb_call(table, tok):
    mesh = plsc.VectorSubcoreMesh(core_axis_name="c", subcore_axis_name="s")
    run = functools.partial(
        pl.kernel,
        mesh=mesh,
        out_type=jax.ShapeDtypeStruct(
            (NUM_WORKERS, NUM_CHUNKS, CHUNK, EMB_D), jnp.float32),
        scratch_types=[
            pltpu.VMEM((NUM_CHUNKS, CHUNK), jnp.int32),
            pltpu.VMEM((CHUNK, EMB_D), jnp.float32),
            pltpu.VMEM((CHUNK, EMB_D), jnp.float32),
            pltpu.SemaphoreType.DMA,
            pltpu.SemaphoreType.DMA,
            pltpu.SemaphoreType.DMA,
            pltpu.SemaphoreType.DMA,
        ],
    )(_emb_body)
    return run(table, tok)


def kernel(tokens, embedding):
    tok = tokens.astype(jnp.int32).reshape(NUM_WORKERS, NUM_CHUNKS, CHUNK)
    out = _emb_call(embedding, tok)
    return out.reshape(tokens.shape[0], tokens.shape[1], EMB_D)
